# baseline (speedup 1.0000x reference)
import functools

import jax
import jax.numpy as jnp
from jax import lax
from jax.experimental import pallas as pl
from jax.experimental.pallas import tpu as pltpu
from jax.experimental.pallas import tpu_sc as plsc

_B, _C, _HW = 64, 768, 196
_A = 16
_BB = 8


def _head_body(x_ref, w_ref, act_ref):
    xb = x_ref[...]
    pooled = jnp.sum(jnp.maximum(xb, 0.0), axis=2) * (1.0 / _HW)
    logits = lax.dot_general(
        pooled, w_ref[...], (((1,), (1,)), ((), ())),
        preferred_element_type=jnp.float32)
    m = jnp.max(logits, axis=1, keepdims=True)
    e = jnp.exp(logits - m)
    p = e / jnp.sum(e, axis=1, keepdims=True)
    idx = lax.broadcasted_iota(jnp.int32, p.shape, 1)
    cand = jnp.where(p >= jnp.max(p, axis=1, keepdims=True), idx, _A)
    act_ref[...] = jnp.min(cand, axis=1, keepdims=True)


def _tc_head(xr, fc1_weight):
    return pl.pallas_call(
        _head_body,
        grid=(_B // _BB,),
        in_specs=[
            pl.BlockSpec((_BB, _C, _HW), lambda i: (i, 0, 0)),
            pl.BlockSpec((_A, _C), lambda i: (0, 0)),
        ],
        out_specs=pl.BlockSpec((_BB, 1), lambda i: (i, 0)),
        out_shape=jax.ShapeDtypeStruct((_B, 1), jnp.int32),
    )(xr, fc1_weight)


_NW_GATHER = 8
_RPW = _B // _NW_GATHER


def _sc_gather(table, idx):
    info = plsc.get_sparse_core_info()
    nc = info.num_cores
    mesh = plsc.VectorSubcoreMesh(core_axis_name="c", subcore_axis_name="s")

    @functools.partial(
        pl.kernel,
        mesh=mesh,
        out_type=jax.ShapeDtypeStruct((_B, _C), jnp.float32),
        scratch_types=[
            pltpu.VMEM((_RPW,), jnp.int32),
            pltpu.VMEM((_RPW, _C), jnp.float32),
            pltpu.SemaphoreType.DMA,
        ],
    )
    def k(table_hbm, idx_hbm, out_hbm, idx_v, rows_v, sem):
        wid = lax.axis_index("s") * nc + lax.axis_index("c")

        @pl.when(wid < _NW_GATHER)
        def _():
            base = wid * _RPW
            pltpu.sync_copy(idx_hbm.at[pl.ds(base, _RPW)], idx_v)
            pltpu.async_copy(table_hbm.at[idx_v], rows_v, sem).wait()
            pltpu.sync_copy(rows_v, out_hbm.at[pl.ds(base, _RPW)])

    return k(table, idx)


def kernel(x, fc1_weight, channel_gates):
    xr = x.reshape(_B, _C, _HW)
    actions2d = _tc_head(xr, fc1_weight)
    actions = actions2d.reshape(_B)
    selected = _sc_gather(channel_gates, actions)
    return actions, selected


# speedup vs baseline: 1.2989x; 1.2989x over previous
import jax
import jax.numpy as jnp
from jax import lax
from jax.experimental import pallas as pl

_B, _C, _HW = 64, 768, 196
_A = 16
_BB = 8


def _head_body(x_ref, w_ref, g_ref, act_ref, sel_ref):
    xb = x_ref[...]
    pooled = jnp.sum(jnp.maximum(xb, 0.0), axis=2) * (1.0 / _HW)
    logits = lax.dot_general(
        pooled, w_ref[...], (((1,), (1,)), ((), ())),
        preferred_element_type=jnp.float32,
        precision=lax.Precision.HIGHEST)
    m = jnp.max(logits, axis=1, keepdims=True)
    e = jnp.exp(logits - m)
    p = e / jnp.sum(e, axis=1, keepdims=True)
    idx = lax.broadcasted_iota(jnp.int32, p.shape, 1)
    cand = jnp.where(p >= jnp.max(p, axis=1, keepdims=True), idx, _A)
    act = jnp.min(cand, axis=1, keepdims=True)
    act_ref[...] = act
    g = g_ref[...]
    sel = jnp.broadcast_to(g[0][None, :], (xb.shape[0], g.shape[1]))
    for a in range(1, _A):
        sel = jnp.where(act == a, g[a][None, :], sel)
    sel_ref[...] = sel


def kernel(x, fc1_weight, channel_gates):
    xr = x.reshape(_B, _C, _HW)
    actions2d, selected = pl.pallas_call(
        _head_body,
        grid=(_B // _BB,),
        in_specs=[
            pl.BlockSpec((_BB, _C, _HW), lambda i: (i, 0, 0)),
            pl.BlockSpec((_A, _C), lambda i: (0, 0)),
            pl.BlockSpec((_A, _C), lambda i: (0, 0)),
        ],
        out_specs=[
            pl.BlockSpec((_BB, 1), lambda i: (i, 0)),
            pl.BlockSpec((_BB, _C), lambda i: (i, 0)),
        ],
        out_shape=[
            jax.ShapeDtypeStruct((_B, 1), jnp.int32),
            jax.ShapeDtypeStruct((_B, _C), jnp.float32),
        ],
    )(xr, fc1_weight, channel_gates)
    return actions2d.reshape(_B), selected
